# baseline (speedup 1.0000x reference)
import functools

import jax
import jax.numpy as jnp
from jax import lax
from jax.experimental import pallas as pl
from jax.experimental.pallas import tpu as pltpu
from jax.experimental.pallas import tpu_sc as plsc

T, C, H = 2048, 768, 3072
NB, EPB = 4, 2
E = NB * EPB
TAU = 1.0
BLK = 256
BLK_SHIFT = 8
G = T // BLK + NB
T_PAD = G * BLK

NC, NS, LANES = 2, 16, 16
NW = NC * NS
RPW = T_PAD // NW
TPW = T // NW


NSEG = NW
SEG = T // NSEG


def _sc_sort_body(ids_hbm, x_hbm, xs_hbm, inv_hbm, tb_hbm,
                  ids_v, cnt2, tab_v, slot_v, tbv, rows_v, cnt_sh, sem):
    cid = lax.axis_index("c")
    sid = lax.axis_index("s")
    g = cid * NS + sid
    l16 = lax.iota(jnp.int32, 16)
    z = jnp.int32(0)

    pltpu.sync_copy(ids_hbm.at[pl.ds(sid * 2 * SEG, 2 * SEG)], ids_v)
    for seg in range(2):
        cnt = (z, z, z, z)
        for ch in range(SEG // LANES):
            v = ids_v[pl.ds((seg * SEG // LANES + ch) * LANES, LANES)]
            cnt = (cnt[0] + jnp.sum((v == 0).astype(jnp.int32)),
                   cnt[1] + jnp.sum((v == 1).astype(jnp.int32)),
                   cnt[2] + jnp.sum((v == 2).astype(jnp.int32)),
                   cnt[3] + jnp.sum((v == 3).astype(jnp.int32)))
        cnt2[pl.ds(seg * LANES, LANES)] = (
            jnp.where(l16 == 0, cnt[0], 0) + jnp.where(l16 == 1, cnt[1], 0)
            + jnp.where(l16 == 2, cnt[2], 0) + jnp.where(l16 == 3, cnt[3], 0))
    pltpu.sync_copy(cnt2, cnt_sh.at[pl.ds(sid * 2 * LANES, 2 * LANES)])
    plsc.subcore_barrier()

    pltpu.sync_copy(cnt_sh, tab_v)
    total = jnp.zeros((LANES,), jnp.int32)
    prefix = jnp.zeros((LANES,), jnp.int32)
    for s in range(NSEG):
        row = tab_v[pl.ds(s * LANES, LANES)]
        total = total + row
        sv = jnp.full((LANES,), s, jnp.int32)
        prefix = prefix + jnp.where(sv < g, row, row * 0)
    padded = ((total + BLK - 1) >> BLK_SHIFT) << BLK_SHIFT
    ends = jnp.cumsum(padded)
    starts = ends - padded
    cursor = starts + prefix
    cur0 = jnp.sum(jnp.where(l16 == 0, cursor, 0))
    cur1 = jnp.sum(jnp.where(l16 == 1, cursor, 0))
    cur2 = jnp.sum(jnp.where(l16 == 2, cursor, 0))
    cur3 = jnp.sum(jnp.where(l16 == 3, cursor, 0))

    @pl.when((sid == 0) & (cid == 0))
    def _():
        e0 = jnp.sum(jnp.where(l16 == 0, ends, 0))
        e1 = jnp.sum(jnp.where(l16 == 1, ends, 0))
        e2 = jnp.sum(jnp.where(l16 == 2, ends, 0))
        e3 = jnp.sum(jnp.where(l16 == 3, ends, 0))
        n_used = e3 >> BLK_SHIFT
        for g16 in range(2):
            lo = (l16 + g16 * LANES) * BLK
            tbx = ((lo >= e0).astype(jnp.int32) + (lo >= e1).astype(jnp.int32)
                   + (lo >= e2).astype(jnp.int32) + (lo >= e3).astype(jnp.int32))
            tbx = jnp.minimum(tbx, NB - 1)
            if g16 == 1:
                tbx = jnp.where(l16 == LANES - 1, n_used, tbx)
            tbv[pl.ds(g16 * LANES, LANES)] = tbx
        pltpu.sync_copy(tbv, tb_hbm)

    pltpu.sync_copy(ids_hbm.at[pl.ds(g * SEG, SEG)], ids_v.at[pl.ds(0, SEG)])
    cur = (cur0, cur1, cur2, cur3)
    for ch in range(SEG // LANES):
        v = ids_v[pl.ds(ch * LANES, LANES)]
        m0 = (v == 0).astype(jnp.int32)
        m1 = (v == 1).astype(jnp.int32)
        m2 = (v == 2).astype(jnp.int32)
        m3 = (v == 3).astype(jnp.int32)
        base = m0 * cur[0] + m1 * cur[1] + m2 * cur[2] + m3 * cur[3]
        rank = (m0 * (jnp.cumsum(m0) - 1) + m1 * (jnp.cumsum(m1) - 1)
                + m2 * (jnp.cumsum(m2) - 1) + m3 * (jnp.cumsum(m3) - 1))
        slot_v[pl.ds(ch * LANES, LANES)] = base + rank
        cur = (cur[0] + jnp.sum(m0), cur[1] + jnp.sum(m1),
               cur[2] + jnp.sum(m2), cur[3] + jnp.sum(m3))

    pltpu.sync_copy(slot_v, inv_hbm.at[pl.ds(g * SEG, SEG)])
    pltpu.sync_copy(x_hbm.at[pl.ds(g * SEG, SEG)], rows_v)
    pltpu.async_copy(rows_v, xs_hbm.at[slot_v], sem).wait()


def _sc_sort_gather(ids, x2):
    mesh = plsc.VectorSubcoreMesh(core_axis_name="c", subcore_axis_name="s")
    return pl.kernel(
        _sc_sort_body,
        out_type=[
            jax.ShapeDtypeStruct((T_PAD, C), jnp.float32),
            jax.ShapeDtypeStruct((T,), jnp.int32),
            jax.ShapeDtypeStruct((NW,), jnp.int32),
        ],
        mesh=mesh,
        compiler_params=pltpu.CompilerParams(needs_layout_passes=False),
        scratch_types=[
            pltpu.VMEM((2 * SEG,), jnp.int32),
            pltpu.VMEM((2 * LANES,), jnp.int32),
            pltpu.VMEM((NSEG * LANES,), jnp.int32),
            pltpu.VMEM((SEG,), jnp.int32),
            pltpu.VMEM((NW,), jnp.int32),
            pltpu.VMEM((SEG, C), jnp.float32),
            pltpu.VMEM_SHARED((NSEG * LANES,), jnp.int32),
            pltpu.SemaphoreType.DMA,
        ],
    )(ids, x2)


def _sc_unsort_body(os_hbm, inv_hbm, out_hbm, inv_v, rows_v, sem):
    cid = lax.axis_index("c")
    sid = lax.axis_index("s")
    wid = cid * NS + sid
    base = wid * TPW
    pltpu.sync_copy(inv_hbm.at[pl.ds(base, TPW)], inv_v)
    pltpu.async_copy(os_hbm.at[inv_v], rows_v, sem).wait()
    pltpu.sync_copy(rows_v, out_hbm.at[pl.ds(base, TPW)])


def _sc_unsort(os, inv):
    mesh = plsc.VectorSubcoreMesh(core_axis_name="c", subcore_axis_name="s")
    return pl.kernel(
        _sc_unsort_body,
        out_type=jax.ShapeDtypeStruct((T, C), jnp.float32),
        mesh=mesh,
        compiler_params=pltpu.CompilerParams(needs_layout_passes=False),
        scratch_types=[
            pltpu.VMEM((TPW,), jnp.int32),
            pltpu.VMEM((TPW, C), jnp.float32),
            pltpu.SemaphoreType.DMA,
        ],
    )(os, inv)


NH = 3
HB = H // NH


def _moe_body(tb_ref, xs_ref, kp_ref, w1_ref, b1_ref, w2_ref, b2_ref,
              sw1_ref, sb1_ref, sw2_ref, sb2_ref, gate_ref, out_ref, acc_ref,
              w1c_ref, w2c_ref):
    j = pl.program_id(0)
    i = pl.program_id(1)

    @pl.when(i < tb_ref[NW - 1])
    def _():
        changed = (i == 0) | (tb_ref[i] != tb_ref[jnp.maximum(i - 1, 0)])

        @pl.when(changed)
        def _():
            w1c_ref[:] = w1_ref[0].astype(jnp.bfloat16)
            w2c_ref[:] = w2_ref[0].astype(jnp.bfloat16)
        xf = xs_ref[:]
        xv = xf.astype(jnp.bfloat16)
        kp = kp_ref[0]
        kn = kp / jnp.maximum(
            jnp.sqrt(jnp.sum(kp * kp, axis=1, keepdims=True)), 1e-12)
        d = (kn[1:2, :] - kn[0:1, :])
        nrm = jnp.maximum(jnp.sqrt(jnp.sum(xf * xf, axis=1, keepdims=True)),
                          1e-12)
        z = lax.dot_general(xf, d, (((1,), (1,)), ((), ())),
                            preferred_element_type=jnp.float32)
        p1 = jax.nn.sigmoid(z / nrm / max(TAU, 1e-6))
        alpha = jax.nn.sigmoid(gate_ref[0])
        h0 = jnp.maximum(
            jnp.dot(xv, w1c_ref[0], preferred_element_type=jnp.float32)
            + b1_ref[0, 0], 0.0).astype(jnp.bfloat16)
        h1 = jnp.maximum(
            jnp.dot(xv, w1c_ref[1], preferred_element_type=jnp.float32)
            + b1_ref[0, 1], 0.0).astype(jnp.bfloat16)
        y0 = jnp.dot(h0, w2c_ref[0], preferred_element_type=jnp.float32)
        y1 = jnp.dot(h1, w2c_ref[1], preferred_element_type=jnp.float32)
        hs = jnp.maximum(
            jnp.dot(xv, sw1_ref[:], preferred_element_type=jnp.float32)
            + sb1_ref[:], 0.0).astype(jnp.bfloat16)
        part = (jnp.dot(hs, sw2_ref[:], preferred_element_type=jnp.float32)
                + alpha * ((1.0 - p1) * y0 + p1 * y1))

        @pl.when(j == 0)
        def _():
            acc_ref[i] = (part + sb2_ref[:]
                          + alpha * ((1.0 - p1) * b2_ref[0, 0]
                                     + p1 * b2_ref[0, 1]))

        @pl.when(j > 0)
        def _():
            tot = acc_ref[i] + part
            acc_ref[i] = tot
            out_ref[:] = tot


def _moe_ffn(tb, xs, kpair, w1p, b1p, w2p, b2p, sW1, sb1, sW2, sb2, gate):
    grid_spec = pltpu.PrefetchScalarGridSpec(
        num_scalar_prefetch=1,
        grid=(NH, G),
        in_specs=[
            pl.BlockSpec((BLK, C), lambda j, i, tb: (i, 0)),
            pl.BlockSpec((1, EPB, C), lambda j, i, tb: (tb[i], 0, 0)),
            pl.BlockSpec((1, EPB, C, HB), lambda j, i, tb: (tb[i], 0, 0, j)),
            pl.BlockSpec((1, EPB, HB), lambda j, i, tb: (tb[i], 0, j)),
            pl.BlockSpec((1, EPB, HB, C), lambda j, i, tb: (tb[i], 0, j, 0)),
            pl.BlockSpec((1, EPB, C), lambda j, i, tb: (tb[i], 0, 0)),
            pl.BlockSpec((C, HB), lambda j, i, tb: (0, j)),
            pl.BlockSpec((1, HB), lambda j, i, tb: (0, j)),
            pl.BlockSpec((HB, C), lambda j, i, tb: (j, 0)),
            pl.BlockSpec((1, C), lambda j, i, tb: (0, 0)),
            pl.BlockSpec(memory_space=pltpu.SMEM),
        ],
        out_specs=pl.BlockSpec((BLK, C), lambda j, i, tb: (i, 0)),
        scratch_shapes=[pltpu.VMEM((G, BLK, C), jnp.float32),
                        pltpu.VMEM((EPB, C, HB), jnp.bfloat16),
                        pltpu.VMEM((EPB, HB, C), jnp.bfloat16)],
    )
    return pl.pallas_call(
        _moe_body,
        grid_spec=grid_spec,
        compiler_params=pltpu.CompilerParams(
            vmem_limit_bytes=63 * 1024 * 1024),
        out_shape=jax.ShapeDtypeStruct((T_PAD, C), jnp.float32),
    )(tb, xs, kpair, w1p, b1p, w2p, b2p, sW1, sb1, sW2, sb2, gate)


def kernel(x, op_id, expert_key, sW1, sb1, sW2, sb2, eW1, eb1, eW2, eb2,
           gate_logit):
    x2 = x.reshape(T, C)
    ids = jnp.clip(op_id.reshape(T).astype(jnp.int32), 0, NB - 1)
    kpair = expert_key.reshape(NB, EPB, C)
    gate = gate_logit.reshape(1)

    xs, inv, tb = _sc_sort_gather(ids, x2)

    w1p = eW1.reshape(NB, EPB, C, H)
    b1p = eb1.reshape(NB, EPB, H)
    w2p = eW2.reshape(NB, EPB, H, C)
    b2p = eb2.reshape(NB, EPB, C)

    os = _moe_ffn(tb, xs, kpair, w1p, b1p, w2p, b2p,
                  sW1.astype(jnp.bfloat16), sb1.reshape(1, H),
                  sW2.astype(jnp.bfloat16), sb2.reshape(1, C), gate)

    out = _sc_unsort(os, inv)
    return out.reshape(x.shape)


# speedup vs baseline: 1.1381x; 1.1381x over previous
import functools

import jax
import jax.numpy as jnp
from jax import lax
from jax.experimental import pallas as pl
from jax.experimental.pallas import tpu as pltpu
from jax.experimental.pallas import tpu_sc as plsc

T, C, H = 2048, 768, 3072
NB, EPB = 4, 2
E = NB * EPB
TAU = 1.0
BLK = 512
BLK_SHIFT = 9
G = T // BLK + NB
T_PAD = G * BLK

NC, NS, LANES = 2, 16, 16
NW = NC * NS
RPW = T_PAD // NW
TPW = T // NW


NSEG = NW
SEG = T // NSEG


def _sc_sort_body(ids_hbm, x_hbm, xs_hbm, inv_hbm, tb_hbm,
                  ids_v, cnt2, tab_v, slot_v, tbv, rows_v, cnt_sh, sem):
    cid = lax.axis_index("c")
    sid = lax.axis_index("s")
    g = cid * NS + sid
    l16 = lax.iota(jnp.int32, 16)
    z = jnp.int32(0)

    pltpu.sync_copy(ids_hbm.at[pl.ds(sid * 2 * SEG, 2 * SEG)], ids_v)
    for seg in range(2):
        cnt = (z, z, z, z)
        for ch in range(SEG // LANES):
            v = ids_v[pl.ds((seg * SEG // LANES + ch) * LANES, LANES)]
            cnt = (cnt[0] + jnp.sum((v == 0).astype(jnp.int32)),
                   cnt[1] + jnp.sum((v == 1).astype(jnp.int32)),
                   cnt[2] + jnp.sum((v == 2).astype(jnp.int32)),
                   cnt[3] + jnp.sum((v == 3).astype(jnp.int32)))
        cnt2[pl.ds(seg * LANES, LANES)] = (
            jnp.where(l16 == 0, cnt[0], 0) + jnp.where(l16 == 1, cnt[1], 0)
            + jnp.where(l16 == 2, cnt[2], 0) + jnp.where(l16 == 3, cnt[3], 0))
    pltpu.sync_copy(cnt2, cnt_sh.at[pl.ds(sid * 2 * LANES, 2 * LANES)])
    plsc.subcore_barrier()

    pltpu.sync_copy(cnt_sh, tab_v)
    total = jnp.zeros((LANES,), jnp.int32)
    prefix = jnp.zeros((LANES,), jnp.int32)
    for s in range(NSEG):
        row = tab_v[pl.ds(s * LANES, LANES)]
        total = total + row
        sv = jnp.full((LANES,), s, jnp.int32)
        prefix = prefix + jnp.where(sv < g, row, row * 0)
    padded = ((total + BLK - 1) >> BLK_SHIFT) << BLK_SHIFT
    ends = jnp.cumsum(padded)
    starts = ends - padded
    cursor = starts + prefix
    cur0 = jnp.sum(jnp.where(l16 == 0, cursor, 0))
    cur1 = jnp.sum(jnp.where(l16 == 1, cursor, 0))
    cur2 = jnp.sum(jnp.where(l16 == 2, cursor, 0))
    cur3 = jnp.sum(jnp.where(l16 == 3, cursor, 0))

    @pl.when((sid == 0) & (cid == 0))
    def _():
        e0 = jnp.sum(jnp.where(l16 == 0, ends, 0))
        e1 = jnp.sum(jnp.where(l16 == 1, ends, 0))
        e2 = jnp.sum(jnp.where(l16 == 2, ends, 0))
        e3 = jnp.sum(jnp.where(l16 == 3, ends, 0))
        n_used = e3 >> BLK_SHIFT
        for g16 in range(2):
            lo = (l16 + g16 * LANES) * BLK
            tbx = ((lo >= e0).astype(jnp.int32) + (lo >= e1).astype(jnp.int32)
                   + (lo >= e2).astype(jnp.int32) + (lo >= e3).astype(jnp.int32))
            tbx = jnp.minimum(tbx, NB - 1)
            if g16 == 1:
                tbx = jnp.where(l16 == LANES - 1, n_used, tbx)
            tbv[pl.ds(g16 * LANES, LANES)] = tbx
        pltpu.sync_copy(tbv, tb_hbm)

    pltpu.sync_copy(ids_hbm.at[pl.ds(g * SEG, SEG)], ids_v.at[pl.ds(0, SEG)])
    cur = (cur0, cur1, cur2, cur3)
    for ch in range(SEG // LANES):
        v = ids_v[pl.ds(ch * LANES, LANES)]
        m0 = (v == 0).astype(jnp.int32)
        m1 = (v == 1).astype(jnp.int32)
        m2 = (v == 2).astype(jnp.int32)
        m3 = (v == 3).astype(jnp.int32)
        base = m0 * cur[0] + m1 * cur[1] + m2 * cur[2] + m3 * cur[3]
        rank = (m0 * (jnp.cumsum(m0) - 1) + m1 * (jnp.cumsum(m1) - 1)
                + m2 * (jnp.cumsum(m2) - 1) + m3 * (jnp.cumsum(m3) - 1))
        slot_v[pl.ds(ch * LANES, LANES)] = base + rank
        cur = (cur[0] + jnp.sum(m0), cur[1] + jnp.sum(m1),
               cur[2] + jnp.sum(m2), cur[3] + jnp.sum(m3))

    pltpu.sync_copy(slot_v, inv_hbm.at[pl.ds(g * SEG, SEG)])
    pltpu.sync_copy(x_hbm.at[pl.ds(g * SEG, SEG)], rows_v)
    pltpu.async_copy(rows_v, xs_hbm.at[slot_v], sem).wait()


def _sc_sort_gather(ids, x2):
    mesh = plsc.VectorSubcoreMesh(core_axis_name="c", subcore_axis_name="s")
    return pl.kernel(
        _sc_sort_body,
        out_type=[
            jax.ShapeDtypeStruct((T_PAD, C), jnp.float32),
            jax.ShapeDtypeStruct((T,), jnp.int32),
            jax.ShapeDtypeStruct((NW,), jnp.int32),
        ],
        mesh=mesh,
        compiler_params=pltpu.CompilerParams(needs_layout_passes=False),
        scratch_types=[
            pltpu.VMEM((2 * SEG,), jnp.int32),
            pltpu.VMEM((2 * LANES,), jnp.int32),
            pltpu.VMEM((NSEG * LANES,), jnp.int32),
            pltpu.VMEM((SEG,), jnp.int32),
            pltpu.VMEM((NW,), jnp.int32),
            pltpu.VMEM((SEG, C), jnp.float32),
            pltpu.VMEM_SHARED((NSEG * LANES,), jnp.int32),
            pltpu.SemaphoreType.DMA,
        ],
    )(ids, x2)


def _sc_unsort_body(os_hbm, inv_hbm, out_hbm, inv_v, rows_v, sem):
    cid = lax.axis_index("c")
    sid = lax.axis_index("s")
    wid = cid * NS + sid
    base = wid * TPW
    pltpu.sync_copy(inv_hbm.at[pl.ds(base, TPW)], inv_v)
    pltpu.async_copy(os_hbm.at[inv_v], rows_v, sem).wait()
    pltpu.sync_copy(rows_v, out_hbm.at[pl.ds(base, TPW)])


def _sc_unsort(os, inv):
    mesh = plsc.VectorSubcoreMesh(core_axis_name="c", subcore_axis_name="s")
    return pl.kernel(
        _sc_unsort_body,
        out_type=jax.ShapeDtypeStruct((T, C), jnp.float32),
        mesh=mesh,
        compiler_params=pltpu.CompilerParams(needs_layout_passes=False),
        scratch_types=[
            pltpu.VMEM((TPW,), jnp.int32),
            pltpu.VMEM((TPW, C), jnp.float32),
            pltpu.SemaphoreType.DMA,
        ],
    )(os, inv)


NH = 3
HB = H // NH


def _moe_body(tb_ref, xs_ref, kp_ref, w1_ref, b1_ref, w2_ref, b2_ref,
              sw1_ref, sb1_ref, sw2_ref, sb2_ref, gate_ref, out_ref, acc_ref):
    j = pl.program_id(0)
    i = pl.program_id(1)

    @pl.when(i < tb_ref[NW - 1])
    def _():
        xf = xs_ref[:]
        xv = xf.astype(jnp.bfloat16)
        kp = kp_ref[0]
        kn = kp / jnp.maximum(
            jnp.sqrt(jnp.sum(kp * kp, axis=1, keepdims=True)), 1e-12)
        d = (kn[1:2, :] - kn[0:1, :])
        nrm = jnp.maximum(jnp.sqrt(jnp.sum(xf * xf, axis=1, keepdims=True)),
                          1e-12)
        z = lax.dot_general(xf, d, (((1,), (1,)), ((), ())),
                            preferred_element_type=jnp.float32)
        p1 = jax.nn.sigmoid(z / nrm / max(TAU, 1e-6))
        alpha = jax.nn.sigmoid(gate_ref[0])
        w1b = w1_ref[0].astype(jnp.bfloat16)
        w2b = w2_ref[0].astype(jnp.bfloat16)
        h0 = jnp.maximum(
            jnp.dot(xv, w1b[0], preferred_element_type=jnp.float32)
            + b1_ref[0, 0], 0.0).astype(jnp.bfloat16)
        h1 = jnp.maximum(
            jnp.dot(xv, w1b[1], preferred_element_type=jnp.float32)
            + b1_ref[0, 1], 0.0).astype(jnp.bfloat16)
        y0 = jnp.dot(h0, w2b[0], preferred_element_type=jnp.float32)
        y1 = jnp.dot(h1, w2b[1], preferred_element_type=jnp.float32)
        hs = jnp.maximum(
            jnp.dot(xv, sw1_ref[:], preferred_element_type=jnp.float32)
            + sb1_ref[:], 0.0).astype(jnp.bfloat16)
        part = (jnp.dot(hs, sw2_ref[:], preferred_element_type=jnp.float32)
                + alpha * ((1.0 - p1) * y0 + p1 * y1))

        @pl.when(j == 0)
        def _():
            acc_ref[i] = (part + sb2_ref[:]
                          + alpha * ((1.0 - p1) * b2_ref[0, 0]
                                     + p1 * b2_ref[0, 1]))

        @pl.when(j > 0)
        def _():
            tot = acc_ref[i] + part
            acc_ref[i] = tot
            out_ref[:] = tot


def _moe_ffn(tb, xs, kpair, w1p, b1p, w2p, b2p, sW1, sb1, sW2, sb2, gate):
    grid_spec = pltpu.PrefetchScalarGridSpec(
        num_scalar_prefetch=1,
        grid=(NH, G),
        in_specs=[
            pl.BlockSpec((BLK, C), lambda j, i, tb: (i, 0)),
            pl.BlockSpec((1, EPB, C), lambda j, i, tb: (tb[i], 0, 0)),
            pl.BlockSpec((1, EPB, C, HB), lambda j, i, tb: (tb[i], 0, 0, j)),
            pl.BlockSpec((1, EPB, HB), lambda j, i, tb: (tb[i], 0, j)),
            pl.BlockSpec((1, EPB, HB, C), lambda j, i, tb: (tb[i], 0, j, 0)),
            pl.BlockSpec((1, EPB, C), lambda j, i, tb: (tb[i], 0, 0)),
            pl.BlockSpec((C, HB), lambda j, i, tb: (0, j)),
            pl.BlockSpec((1, HB), lambda j, i, tb: (0, j)),
            pl.BlockSpec((HB, C), lambda j, i, tb: (j, 0)),
            pl.BlockSpec((1, C), lambda j, i, tb: (0, 0)),
            pl.BlockSpec(memory_space=pltpu.SMEM),
        ],
        out_specs=pl.BlockSpec((BLK, C), lambda j, i, tb: (i, 0)),
        scratch_shapes=[pltpu.VMEM((G, BLK, C), jnp.float32)],
    )
    return pl.pallas_call(
        _moe_body,
        grid_spec=grid_spec,
        compiler_params=pltpu.CompilerParams(
            vmem_limit_bytes=63 * 1024 * 1024),
        out_shape=jax.ShapeDtypeStruct((T_PAD, C), jnp.float32),
    )(tb, xs, kpair, w1p, b1p, w2p, b2p, sW1, sb1, sW2, sb2, gate)


def kernel(x, op_id, expert_key, sW1, sb1, sW2, sb2, eW1, eb1, eW2, eb2,
           gate_logit):
    x2 = x.reshape(T, C)
    ids = jnp.clip(op_id.reshape(T).astype(jnp.int32), 0, NB - 1)
    kpair = expert_key.reshape(NB, EPB, C)
    gate = gate_logit.reshape(1)

    xs, inv, tb = _sc_sort_gather(ids, x2)

    w1p = eW1.reshape(NB, EPB, C, H)
    b1p = eb1.reshape(NB, EPB, H)
    w2p = eW2.reshape(NB, EPB, H, C)
    b2p = eb2.reshape(NB, EPB, C)

    os = _moe_ffn(tb, xs, kpair, w1p, b1p, w2p, b2p,
                  sW1.astype(jnp.bfloat16), sb1.reshape(1, H),
                  sW2.astype(jnp.bfloat16), sb2.reshape(1, C), gate)

    out = _sc_unsort(os, inv)
    return out.reshape(x.shape)
